# baseline (speedup 1.0000x reference)
import functools

import jax
import jax.numpy as jnp
from jax import lax
from jax.experimental import pallas as pl
from jax.experimental.pallas import tpu as pltpu
from jax.experimental.pallas import tpu_sc as plsc

NUM_ROWS = 128
NUM_COLS = 32768
LANES = 16
CHUNKS = NUM_COLS // LANES
NUM_WORKERS = 32
ROWS_PER_WORKER = NUM_ROWS // NUM_WORKERS
U = 8
INT_MAX = 2**31 - 1
SLOT = 16


def _argmax_row(xrow, lane_iota):
    def scan_body(i, accs):
        out = []
        base = i * (U * LANES)
        bi = jnp.full((LANES,), 0, jnp.int32) + i
        for u in range(U):
            vmax, viter = accs[2 * u], accs[2 * u + 1]
            v = xrow[pl.ds(base + u * LANES, LANES)]
            cond = v > vmax
            out.append(jnp.where(cond, v, vmax))
            out.append(jnp.where(cond, bi, viter))
        return tuple(out)

    init = []
    for _ in range(U):
        init.append(jnp.full((LANES,), -jnp.inf, jnp.float32))
        init.append(jnp.zeros((LANES,), jnp.int32))
    accs = lax.fori_loop(0, CHUNKS // U, scan_body, tuple(init))

    best_v = accs[0]
    best_i = accs[1] * (U * LANES) + lane_iota
    for u in range(1, U):
        v = accs[2 * u]
        idx = accs[2 * u + 1] * (U * LANES) + (u * LANES) + lane_iota
        better = (v > best_v) | ((v == best_v) & (idx < best_i))
        best_v = jnp.where(better, v, best_v)
        best_i = jnp.where(better, idx, best_i)

    gmax = jnp.max(best_v)
    cand = jnp.where(best_v == gmax, best_i, jnp.int32(INT_MAX))
    return jnp.min(cand)


def _sc_body(x_hbm, idx_hbm, xbuf0, xbuf1, idx_buf, sem_in, sem_out):
    xbufs = [xbuf0, xbuf1]
    wid = lax.axis_index("s") * 2 + lax.axis_index("c")
    row0 = wid * ROWS_PER_WORKER

    in_copy = pltpu.async_copy(x_hbm.at[row0], xbuf0, sem_in)
    lane_iota = lax.broadcasted_iota(jnp.int32, (LANES,), 0)

    idxs = []
    for r in range(ROWS_PER_WORKER):
        in_copy.wait()
        if r + 1 < ROWS_PER_WORKER:
            in_copy = pltpu.async_copy(
                x_hbm.at[row0 + r + 1], xbufs[(r + 1) % 2], sem_in)
        idxs.append(_argmax_row(xbufs[r % 2], lane_iota))

    vec = jnp.zeros((LANES,), jnp.int32)
    for r in range(ROWS_PER_WORKER):
        vec = jnp.where(lane_iota == r, idxs[r], vec)
    idx_buf[...] = vec
    pltpu.async_copy(idx_buf, idx_hbm.at[pl.ds(wid * SLOT, SLOT)],
                     sem_out).wait()


@jax.jit
def _hardmax_idx_sc(x):
    mesh = plsc.VectorSubcoreMesh(core_axis_name="c", subcore_axis_name="s",
                                  num_cores=2, num_subcores=16)
    return pl.kernel(
        _sc_body,
        out_type=jax.ShapeDtypeStruct((NUM_WORKERS * SLOT,), jnp.int32),
        mesh=mesh,
        scratch_types=[
            pltpu.VMEM((NUM_COLS,), jnp.float32),
            pltpu.VMEM((NUM_COLS,), jnp.float32),
            pltpu.VMEM((SLOT,), jnp.int32),
            pltpu.SemaphoreType.DMA,
            pltpu.SemaphoreType.DMA,
        ],
        compiler_params=pltpu.CompilerParams(needs_layout_passes=False),
    )(x)


ROWS_PER_BLOCK = 8
COLS_PER_BLOCK = 4096


def _tc_body(idx_smem, out_ref):
    s = pl.program_id(0)
    j = pl.program_id(1)
    col0 = j * COLS_PER_BLOCK
    iota = col0 + lax.broadcasted_iota(jnp.int32, (1, COLS_PER_BLOCK), 1)
    for rr in range(ROWS_PER_BLOCK):
        idx = idx_smem[s * ROWS_PER_BLOCK + rr]
        out_ref[rr, :] = (iota == idx).astype(jnp.int32)[0, :]


@jax.jit
def _onehot_tc(idx):
    grid_spec = pltpu.PrefetchScalarGridSpec(
        num_scalar_prefetch=1,
        grid=(NUM_ROWS // ROWS_PER_BLOCK, NUM_COLS // COLS_PER_BLOCK),
        in_specs=[],
        out_specs=pl.BlockSpec((ROWS_PER_BLOCK, COLS_PER_BLOCK),
                               lambda s, j, *_: (s, j)),
    )
    return pl.pallas_call(
        _tc_body,
        grid_spec=grid_spec,
        out_shape=jax.ShapeDtypeStruct((NUM_ROWS, NUM_COLS), jnp.int32),
        compiler_params=pltpu.CompilerParams(
            dimension_semantics=("parallel", "arbitrary")),
    )(idx)


def kernel(x):
    packed = _hardmax_idx_sc(x)
    idx = packed.reshape(NUM_WORKERS, SLOT)[:, :ROWS_PER_WORKER].reshape(
        NUM_ROWS)
    return _onehot_tc(idx)


# speedup vs baseline: 1.9963x; 1.9963x over previous
import functools

import jax
import jax.numpy as jnp
from jax import lax
from jax.experimental import pallas as pl
from jax.experimental.pallas import tpu as pltpu
from jax.experimental.pallas import tpu_sc as plsc

NUM_ROWS = 128
NUM_COLS = 32768
LANES = 16
CHUNKS = NUM_COLS // LANES
NUM_WORKERS = 32
ROWS_PER_WORKER = NUM_ROWS // NUM_WORKERS
U = 8
INT_MAX = 2**31 - 1
SLOT = 16


def _argmax_row(xrow, lane_iota):
    def scan_body(i, accs):
        out = []
        base = i * (U * LANES)
        bi = jnp.full((LANES,), 0, jnp.int32) + i
        for u in range(U):
            vmax, viter = accs[2 * u], accs[2 * u + 1]
            v = xrow[pl.ds(base + u * LANES, LANES)]
            cond = v > vmax
            out.append(jnp.where(cond, v, vmax))
            out.append(jnp.where(cond, bi, viter))
        return tuple(out)

    init = []
    for _ in range(U):
        init.append(jnp.full((LANES,), -jnp.inf, jnp.float32))
        init.append(jnp.zeros((LANES,), jnp.int32))
    accs = lax.fori_loop(0, CHUNKS // U, scan_body, tuple(init))

    best_v = accs[0]
    best_i = accs[1] * (U * LANES) + lane_iota
    for u in range(1, U):
        v = accs[2 * u]
        idx = accs[2 * u + 1] * (U * LANES) + (u * LANES) + lane_iota
        better = (v > best_v) | ((v == best_v) & (idx < best_i))
        best_v = jnp.where(better, v, best_v)
        best_i = jnp.where(better, idx, best_i)

    gmax = jnp.max(best_v)
    cand = jnp.where(best_v == gmax, best_i, jnp.int32(INT_MAX))
    return jnp.min(cand)


def _sc_body(x_hbm, idx_hbm, xbuf0, xbuf1, idx_buf, sem_in, sem_out):
    xbufs = [xbuf0, xbuf1]
    wid = lax.axis_index("s") * 2 + lax.axis_index("c")
    row0 = wid * ROWS_PER_WORKER

    in_copy = pltpu.async_copy(x_hbm.at[row0], xbuf0, sem_in)
    lane_iota = lax.broadcasted_iota(jnp.int32, (LANES,), 0)

    for r in range(ROWS_PER_WORKER):
        in_copy.wait()
        if r + 1 < ROWS_PER_WORKER:
            in_copy = pltpu.async_copy(
                x_hbm.at[row0 + r + 1], xbufs[(r + 1) % 2], sem_in)
        idx = _argmax_row(xbufs[r % 2], lane_iota)
        idx_buf[r, :] = jnp.zeros((LANES,), jnp.int32) + idx

    pltpu.async_copy(idx_buf, idx_hbm.at[pl.ds(row0, ROWS_PER_WORKER)],
                     sem_out).wait()


@jax.jit
def _hardmax_idx_sc(x):
    mesh = plsc.VectorSubcoreMesh(core_axis_name="c", subcore_axis_name="s",
                                  num_cores=2, num_subcores=16)
    return pl.kernel(
        _sc_body,
        out_type=jax.ShapeDtypeStruct((NUM_ROWS, SLOT), jnp.int32),
        mesh=mesh,
        scratch_types=[
            pltpu.VMEM((NUM_COLS,), jnp.float32),
            pltpu.VMEM((NUM_COLS,), jnp.float32),
            pltpu.VMEM((ROWS_PER_WORKER, SLOT), jnp.int32),
            pltpu.SemaphoreType.DMA,
            pltpu.SemaphoreType.DMA,
        ],
        compiler_params=pltpu.CompilerParams(needs_layout_passes=False),
    )(x)


ROWS_PER_BLOCK = 8
COLS_PER_BLOCK = 32768


def _tc_body(idx_ref, out_ref):
    j = pl.program_id(1)
    col0 = j * COLS_PER_BLOCK
    iota = col0 + lax.broadcasted_iota(
        jnp.int32, (ROWS_PER_BLOCK, COLS_PER_BLOCK), 1)
    out_ref[...] = (iota == idx_ref[:, 0:1]).astype(jnp.int32)


@jax.jit
def _onehot_tc(idx):
    return pl.pallas_call(
        _tc_body,
        grid=(NUM_ROWS // ROWS_PER_BLOCK, NUM_COLS // COLS_PER_BLOCK),
        in_specs=[pl.BlockSpec((ROWS_PER_BLOCK, SLOT), lambda s, j: (s, 0))],
        out_specs=pl.BlockSpec((ROWS_PER_BLOCK, COLS_PER_BLOCK),
                               lambda s, j: (s, j)),
        out_shape=jax.ShapeDtypeStruct((NUM_ROWS, NUM_COLS), jnp.int32),
        compiler_params=pltpu.CompilerParams(
            dimension_semantics=("parallel", "arbitrary")),
    )(idx)


def kernel(x):
    return _onehot_tc(_hardmax_idx_sc(x))
